# baseline (speedup 1.0000x reference)
import functools

import jax
import jax.numpy as jnp
from jax import lax
from jax.experimental import pallas as pl
from jax.experimental.pallas import tpu as pltpu
from jax.experimental.pallas import tpu_sc as plsc

_N, _D = 100000, 256
_BR = 4000
_NT = 92000
_NS = _N - _NT

_R = 80
_NCHUNK = _N // _R
_C0 = _NT // _R
_NW = 32
_L = 16
_SC_CHUNKS = _NS // _R
_T = -2 * (-_SC_CHUNKS // (2 * _NW))

_mesh = plsc.VectorSubcoreMesh(core_axis_name="c", subcore_axis_name="s")


@functools.partial(
    pl.kernel,
    out_type=jax.ShapeDtypeStruct((_N, _D), jnp.float32),
    mesh=_mesh,
    scratch_types=[
        pltpu.VMEM((2, _R, _D), jnp.float32),
        pltpu.VMEM((2, _R, _D), jnp.float32),
        pltpu.VMEM((2, _R, _D), jnp.float32),
        pltpu.VMEM((2, _R), jnp.float32),
        pltpu.VMEM((2, _R), jnp.float32),
        pltpu.SemaphoreType.DMA,
        pltpu.SemaphoreType.DMA,
        pltpu.SemaphoreType.DMA,
        pltpu.SemaphoreType.DMA,
    ],
)
def _sc_kernel(x_hbm, a_hbm, h_hbm, w_hbm, o_hbm,
               xb, hb, ob, ab, wb, in0, in1, out0, out1):
    wid = lax.axis_index("s") * 2 + lax.axis_index("c")
    insem = (in0, in1)
    outsem = (out0, out1)

    def cid(i):
        return wid + i * _NW

    def in_copies(i, b):
        c = cid(i) + _C0
        row0 = c * _R
        return (
            pltpu.make_async_copy(x_hbm.at[pl.ds(row0, _R)], xb.at[b], insem[b]),
            pltpu.make_async_copy(h_hbm.at[pl.ds(row0, _R)], hb.at[b], insem[b]),
            pltpu.make_async_copy(a_hbm.at[pl.ds(row0, _R)], ab.at[b], insem[b]),
            pltpu.make_async_copy(w_hbm.at[pl.ds(row0, _R)], wb.at[b], insem[b]),
        )

    def out_copy(i, b):
        row0 = (cid(i) + _C0) * _R
        return pltpu.make_async_copy(ob.at[b], o_hbm.at[pl.ds(row0, _R)],
                                     outsem[b])

    def start_in(i, b):
        @pl.when(cid(i) < _SC_CHUNKS)
        def _():
            for cp in in_copies(i, b):
                cp.start()

    def wait_in(i, b):
        @pl.when(cid(i) < _SC_CHUNKS)
        def _():
            for cp in in_copies(i, b):
                cp.wait()

    def start_out(i, b):
        @pl.when(cid(i) < _SC_CHUNKS)
        def _():
            out_copy(i, b).start()

    def wait_out(i, b):
        @pl.when((i >= 0) & (cid(i) < _SC_CHUNKS))
        def _():
            out_copy(i, b).wait()

    def compute(i, b):
        @pl.when(cid(i) < _SC_CHUNKS)
        def _():
            def grp_body(g, rc):
                sl = pl.ds(g * _L, _L)
                bv = jnp.clip(1.0 - wb[b, sl] * ab[b, sl], 0.0, 1.0)
                for k in range(_L):
                    beta = bv[k]
                    r = g * _L + k
                    for j in range(_D // _L):
                        cs = pl.ds(j * _L, _L)
                        ob[b, r, cs] = hb[b, r, cs] * beta + xb[b, r, cs]
                return rc

            lax.fori_loop(0, _R // _L, grp_body, 0)

    def step(i, b):
        start_in(i + 1, 1 - b)
        wait_in(i, b)
        wait_out(i - 2, b)
        compute(i, b)
        start_out(i, b)

    start_in(0, 0)

    def pair(p, carry):
        step(2 * p, 0)
        step(2 * p + 1, 1)
        return carry

    lax.fori_loop(0, _T // 2, pair, 0)
    wait_out(_T - 2, 0)
    wait_out(_T - 1, 1)


def _tc_body(x_ref, a_ref, h_ref, w_ref, prev_ref, o_ref):
    del prev_ref
    beta = jnp.clip(1.0 - w_ref[0] * a_ref[0], 0.0, 1.0)
    beta = beta.reshape(_BR, 1)
    o_ref[...] = h_ref[...] * beta + x_ref[...]


def _tc_part(x, a3, his_x, w3, prev_out):
    return pl.pallas_call(
        _tc_body,
        grid=(_NT // _BR,),
        in_specs=[
            pl.BlockSpec((_BR, _D), lambda i: (i, 0)),
            pl.BlockSpec((1, 1, _BR), lambda i: (i, 0, 0)),
            pl.BlockSpec((_BR, _D), lambda i: (i, 0)),
            pl.BlockSpec((1, 1, _BR), lambda i: (i, 0, 0)),
            pl.BlockSpec(memory_space=pl.ANY),
        ],
        out_specs=pl.BlockSpec((_BR, _D), lambda i: (i, 0)),
        out_shape=jax.ShapeDtypeStruct((_N, _D), jnp.float32),
        input_output_aliases={4: 0},
    )(x, a3, his_x, w3, prev_out)


def kernel(x, agg_n, his_x, inv_w):
    a3 = agg_n.reshape(_N // _BR, 1, _BR)
    w3 = inv_w.reshape(_N // _BR, 1, _BR)
    sc_out = _sc_kernel(x, agg_n, his_x, inv_w)
    return _tc_part(x, a3, his_x, w3, sc_out)


# speedup vs baseline: 1.0520x; 1.0520x over previous
import functools

import jax
import jax.numpy as jnp
from jax import lax
from jax.experimental import pallas as pl
from jax.experimental.pallas import tpu as pltpu
from jax.experimental.pallas import tpu_sc as plsc

_N, _D = 100000, 256
_BR = 4000
_NT = 92000
_NS = _N - _NT

_R = 80
_NCHUNK = _N // _R
_C0 = _NT // _R
_NW = 32
_L = 16
_SC_CHUNKS = _NS // _R
_T = -2 * (-_SC_CHUNKS // (2 * _NW))

_mesh = plsc.VectorSubcoreMesh(core_axis_name="c", subcore_axis_name="s")


@functools.partial(
    pl.kernel,
    out_type=jax.ShapeDtypeStruct((_NS, _D), jnp.float32),
    mesh=_mesh,
    scratch_types=[
        pltpu.VMEM((2, _R, _D), jnp.float32),
        pltpu.VMEM((2, _R, _D), jnp.float32),
        pltpu.VMEM((2, _R, _D), jnp.float32),
        pltpu.VMEM((2, _R), jnp.float32),
        pltpu.VMEM((2, _R), jnp.float32),
        pltpu.SemaphoreType.DMA,
        pltpu.SemaphoreType.DMA,
        pltpu.SemaphoreType.DMA,
        pltpu.SemaphoreType.DMA,
    ],
)
def _sc_kernel(x_hbm, a_hbm, h_hbm, w_hbm, o_hbm,
               xb, hb, ob, ab, wb, in0, in1, out0, out1):
    wid = lax.axis_index("s") * 2 + lax.axis_index("c")
    insem = (in0, in1)
    outsem = (out0, out1)

    def cid(i):
        return wid + i * _NW

    def in_copies(i, b):
        c = cid(i) + _C0
        row0 = c * _R
        return (
            pltpu.make_async_copy(x_hbm.at[pl.ds(row0, _R)], xb.at[b], insem[b]),
            pltpu.make_async_copy(h_hbm.at[pl.ds(row0, _R)], hb.at[b], insem[b]),
            pltpu.make_async_copy(a_hbm.at[pl.ds(row0, _R)], ab.at[b], insem[b]),
            pltpu.make_async_copy(w_hbm.at[pl.ds(row0, _R)], wb.at[b], insem[b]),
        )

    def out_copy(i, b):
        row0 = cid(i) * _R
        return pltpu.make_async_copy(ob.at[b], o_hbm.at[pl.ds(row0, _R)],
                                     outsem[b])

    def start_in(i, b):
        @pl.when(cid(i) < _SC_CHUNKS)
        def _():
            for cp in in_copies(i, b):
                cp.start()

    def wait_in(i, b):
        @pl.when(cid(i) < _SC_CHUNKS)
        def _():
            for cp in in_copies(i, b):
                cp.wait()

    def start_out(i, b):
        @pl.when(cid(i) < _SC_CHUNKS)
        def _():
            out_copy(i, b).start()

    def wait_out(i, b):
        @pl.when((i >= 0) & (cid(i) < _SC_CHUNKS))
        def _():
            out_copy(i, b).wait()

    def compute(i, b):
        @pl.when(cid(i) < _SC_CHUNKS)
        def _():
            def grp_body(g, rc):
                sl = pl.ds(g * _L, _L)
                bv = jnp.clip(1.0 - wb[b, sl] * ab[b, sl], 0.0, 1.0)
                for k in range(_L):
                    beta = bv[k]
                    r = g * _L + k
                    for j in range(_D // _L):
                        cs = pl.ds(j * _L, _L)
                        ob[b, r, cs] = hb[b, r, cs] * beta + xb[b, r, cs]
                return rc

            lax.fori_loop(0, _R // _L, grp_body, 0)

    def step(i, b):
        start_in(i + 1, 1 - b)
        wait_in(i, b)
        wait_out(i - 2, b)
        compute(i, b)
        start_out(i, b)

    start_in(0, 0)

    def pair(p, carry):
        step(2 * p, 0)
        step(2 * p + 1, 1)
        return carry

    lax.fori_loop(0, _T // 2, pair, 0)
    wait_out(_T - 2, 0)
    wait_out(_T - 1, 1)


def _tc_body(x_ref, a_ref, h_ref, w_ref, o_ref):
    beta = jnp.clip(1.0 - w_ref[0] * a_ref[0], 0.0, 1.0)
    beta = beta.reshape(_BR, 1)
    o_ref[...] = h_ref[...] * beta + x_ref[...]


def _tc_part(x, a3, his_x, w3):
    return pl.pallas_call(
        _tc_body,
        grid=(_NT // _BR,),
        in_specs=[
            pl.BlockSpec((_BR, _D), lambda i: (i, 0)),
            pl.BlockSpec((1, 1, _BR), lambda i: (i, 0, 0)),
            pl.BlockSpec((_BR, _D), lambda i: (i, 0)),
            pl.BlockSpec((1, 1, _BR), lambda i: (i, 0, 0)),
        ],
        out_specs=pl.BlockSpec((_BR, _D), lambda i: (i, 0)),
        out_shape=jax.ShapeDtypeStruct((_N, _D), jnp.float32),
    )(x, a3, his_x, w3)


def kernel(x, agg_n, his_x, inv_w):
    a3 = agg_n.reshape(_N // _BR, 1, _BR)
    w3 = inv_w.reshape(_N // _BR, 1, _BR)
    sc_part = _sc_kernel(x, agg_n, his_x, inv_w)
    tc_full = _tc_part(x, a3, his_x, w3)
    return lax.dynamic_update_slice(tc_full, sc_part, (_NT, 0))


# speedup vs baseline: 1.3632x; 1.2958x over previous
import jax
import jax.numpy as jnp
from jax.experimental import pallas as pl

_N, _D = 100000, 256
_BR = 4096


def _body(x_ref, a_ref, h_ref, w_ref, o_ref):
    beta = jnp.clip(1.0 - w_ref[...] * a_ref[...], 0.0, 1.0)
    beta = beta.reshape(_BR, 1)
    o_ref[...] = h_ref[...] * beta + x_ref[...]


def kernel(x, agg_n, his_x, inv_w):
    return pl.pallas_call(
        _body,
        grid=((_N + _BR - 1) // _BR,),
        in_specs=[
            pl.BlockSpec((_BR, _D), lambda i: (i, 0)),
            pl.BlockSpec((_BR,), lambda i: (i,)),
            pl.BlockSpec((_BR, _D), lambda i: (i, 0)),
            pl.BlockSpec((_BR,), lambda i: (i,)),
        ],
        out_specs=pl.BlockSpec((_BR, _D), lambda i: (i, 0)),
        out_shape=jax.ShapeDtypeStruct((_N, _D), jnp.float32),
    )(x, agg_n, his_x, inv_w)
